# baseline (speedup 1.0000x reference)
import functools

import jax
import jax.numpy as jnp
from jax.experimental import pallas as pl
from jax.experimental.pallas import tpu as pltpu

NUM_CLAUSES = 8192
NUM_LITS = 4096
NUM_VARS = NUM_LITS // 2
D = 128
NUM_ROUNDS = 4

BC = 1024
BL = 512


def _fdot(a, b):
    return jnp.dot(a, b, preferred_element_type=jnp.float32,
                   precision=jax.lax.Precision.HIGHEST)


def _clause_body(g_ref, l_ref, c_ref, ws_ref, wm_ref, b1_ref, w2_ref, b2_ref,
                 o_ref):
    msgs = jnp.dot(g_ref[...], l_ref[...], preferred_element_type=jnp.float32)
    h = jnp.maximum(
        _fdot(msgs, wm_ref[...]) + _fdot(c_ref[...], ws_ref[...]) +
        b1_ref[...], 0.0)
    o_ref[...] = _fdot(h, w2_ref[...]) + b2_ref[...]


def _literal_body(g_ref, c_ref, l_ref, lf_ref, ws_ref, wm_ref, wf_ref, b1_ref,
                  w2_ref, b2_ref, o_ref):
    msgs = jax.lax.dot_general(
        g_ref[...], c_ref[...],
        dimension_numbers=(((0,), (0,)), ((), ())),
        preferred_element_type=jnp.float32)
    h = jnp.maximum(
        _fdot(msgs, wm_ref[...]) + _fdot(l_ref[...], ws_ref[...]) +
        _fdot(lf_ref[...], wf_ref[...]) + b1_ref[...], 0.0)
    o_ref[...] = _fdot(h, w2_ref[...]) + b2_ref[...]


def _vote_body(l_ref, w1a_ref, w1b_ref, b1_ref, w2_ref, b2_ref, w3_ref,
               b3_ref, w4_ref, b4_ref, o_ref):
    v1 = l_ref[:NUM_VARS, :]
    v2 = l_ref[NUM_VARS:, :]
    h = jnp.maximum(
        _fdot(v1, w1a_ref[...]) + _fdot(v2, w1b_ref[...]) + b1_ref[...], 0.0)
    h = jnp.maximum(_fdot(h, w2_ref[...]) + b2_ref[...], 0.0)
    h = jnp.maximum(_fdot(h, w3_ref[...]) + b3_ref[...], 0.0)
    o_ref[...] = _fdot(h, w4_ref[...]) + b4_ref[...]


def _clause_phase(g_bf, l_bf, c, ws, wm, b1, w2, b2):
    nblk = NUM_CLAUSES // BC
    return pl.pallas_call(
        _clause_body,
        grid=(nblk,),
        in_specs=[
            pl.BlockSpec((BC, NUM_LITS), lambda i: (i, 0)),
            pl.BlockSpec((NUM_LITS, D), lambda i: (0, 0)),
            pl.BlockSpec((BC, D), lambda i: (i, 0)),
            pl.BlockSpec((D, D), lambda i: (0, 0)),
            pl.BlockSpec((D, D), lambda i: (0, 0)),
            pl.BlockSpec((1, D), lambda i: (0, 0)),
            pl.BlockSpec((D, D), lambda i: (0, 0)),
            pl.BlockSpec((1, D), lambda i: (0, 0)),
        ],
        out_specs=pl.BlockSpec((BC, D), lambda i: (i, 0)),
        out_shape=jax.ShapeDtypeStruct((NUM_CLAUSES, D), jnp.float32),
    )(g_bf, l_bf, c, ws, wm, b1, w2, b2)


def _literal_phase(g_bf, c_bf, l, ws, wm, wf, b1, w2, b2):
    nblk = NUM_LITS // BL
    half = nblk // 2
    return pl.pallas_call(
        _literal_body,
        grid=(nblk,),
        in_specs=[
            pl.BlockSpec((NUM_CLAUSES, BL), lambda j: (0, j)),
            pl.BlockSpec((NUM_CLAUSES, D), lambda j: (0, 0)),
            pl.BlockSpec((BL, D), lambda j: (j, 0)),
            pl.BlockSpec((BL, D), lambda j: ((j + half) % nblk, 0)),
            pl.BlockSpec((D, D), lambda j: (0, 0)),
            pl.BlockSpec((D, D), lambda j: (0, 0)),
            pl.BlockSpec((D, D), lambda j: (0, 0)),
            pl.BlockSpec((1, D), lambda j: (0, 0)),
            pl.BlockSpec((D, D), lambda j: (0, 0)),
            pl.BlockSpec((1, D), lambda j: (0, 0)),
        ],
        out_specs=pl.BlockSpec((BL, D), lambda j: (j, 0)),
        out_shape=jax.ShapeDtypeStruct((NUM_LITS, D), jnp.float32),
    )(g_bf, c_bf, l, l, ws, wm, wf, b1, w2, b2)


def _vote_phase(l, v_params):
    (w1, b1), (w2, b2), (w3, b3), (w4, b4) = v_params
    w1a, w1b = w1[:D], w1[D:]
    args = (l, w1a, w1b, b1.reshape(1, D), w2, b2.reshape(1, D), w3,
            b3.reshape(1, D), w4, b4.reshape(1, 1))
    return pl.pallas_call(
        _vote_body,
        in_specs=[
            pl.BlockSpec(a.shape, functools.partial(lambda n: (0,) * n, a.ndim))
            for a in args
        ],
        out_specs=pl.BlockSpec((NUM_VARS, 1), lambda: (0, 0)),
        out_shape=jax.ShapeDtypeStruct((NUM_VARS, 1), jnp.float32),
    )(*args)


def kernel(G, c_params, l_params, v_params, c_init_scale, l_init_scale,
           cl_scale, lc_scale):
    g_bf = G.astype(jnp.bfloat16)

    L = jnp.full((NUM_LITS, D), 1.0, jnp.float32) * l_init_scale
    C = jnp.full((NUM_CLAUSES, D), 1.0, jnp.float32) * c_init_scale

    for r in range(NUM_ROUNDS):
        (w1c, b1c), (w2c, b2c) = c_params[r]
        C = _clause_phase(g_bf, L.astype(jnp.bfloat16), C,
                          w1c[:D], w1c[D:] * lc_scale, b1c.reshape(1, D),
                          w2c, b2c.reshape(1, D))
        (w1l, b1l), (w2l, b2l) = l_params[r]
        L = _literal_phase(g_bf, C.astype(jnp.bfloat16), L,
                           w1l[:D], w1l[D:2 * D] * cl_scale, w1l[2 * D:],
                           b1l.reshape(1, D), w2l, b2l.reshape(1, D))

    return _vote_phase(L, v_params).reshape(NUM_VARS)


# speedup vs baseline: 1.1034x; 1.1034x over previous
import functools

import jax
import jax.numpy as jnp
from jax.experimental import pallas as pl
from jax.experimental.pallas import tpu as pltpu

NUM_CLAUSES = 8192
NUM_LITS = 4096
NUM_VARS = NUM_LITS // 2
D = 128
NUM_ROUNDS = 4

BC = 1024
BL = 512


def _dot(a, b):
    return jnp.dot(a, b, preferred_element_type=jnp.float32)


def _clause_body(g_ref, l_ref, c_ref, w1_ref, b1_ref, w2_ref, b2_ref,
                 sc_ref, o_ref):
    msgs = _dot(g_ref[...], l_ref[...]) * sc_ref[0]
    x = jnp.concatenate([c_ref[...], msgs], axis=1)
    h = jnp.maximum(_dot(x, w1_ref[...]) + b1_ref[...], 0.0)
    o_ref[...] = _dot(h, w2_ref[...]) + b2_ref[...]


def _literal_body(g_ref, c_ref, l_ref, lf_ref, w1_ref, b1_ref, w2_ref,
                  b2_ref, sc_ref, o_ref):
    msgs = jax.lax.dot_general(
        g_ref[...], c_ref[...],
        dimension_numbers=(((0,), (0,)), ((), ())),
        preferred_element_type=jnp.float32) * sc_ref[0]
    x = jnp.concatenate([l_ref[...], msgs, lf_ref[...]], axis=1)
    h = jnp.maximum(_dot(x, w1_ref[...]) + b1_ref[...], 0.0)
    o_ref[...] = _dot(h, w2_ref[...]) + b2_ref[...]


def _vote_body(l_ref, w1_ref, b1_ref, w2_ref, b2_ref, w3_ref,
               b3_ref, w4_ref, b4_ref, o_ref):
    v = jnp.concatenate([l_ref[:NUM_VARS, :], l_ref[NUM_VARS:, :]], axis=1)
    h = jnp.maximum(_dot(v, w1_ref[...]) + b1_ref[...], 0.0)
    h = jnp.maximum(_dot(h, w2_ref[...]) + b2_ref[...], 0.0)
    h = jnp.maximum(_dot(h, w3_ref[...]) + b3_ref[...], 0.0)
    o_ref[...] = _dot(h, w4_ref[...]) + b4_ref[...]


def _clause_phase(g, l, c, w1, b1, w2, b2, scale):
    nblk = NUM_CLAUSES // BC
    return pl.pallas_call(
        _clause_body,
        grid=(nblk,),
        in_specs=[
            pl.BlockSpec((BC, NUM_LITS), lambda i: (i, 0)),
            pl.BlockSpec((NUM_LITS, D), lambda i: (0, 0)),
            pl.BlockSpec((BC, D), lambda i: (i, 0)),
            pl.BlockSpec((2 * D, D), lambda i: (0, 0)),
            pl.BlockSpec((1, D), lambda i: (0, 0)),
            pl.BlockSpec((D, D), lambda i: (0, 0)),
            pl.BlockSpec((1, D), lambda i: (0, 0)),
            pl.BlockSpec(memory_space=pltpu.SMEM),
        ],
        out_specs=pl.BlockSpec((BC, D), lambda i: (i, 0)),
        out_shape=jax.ShapeDtypeStruct((NUM_CLAUSES, D), jnp.float32),
    )(g, l, c, w1, b1, w2, b2, scale)


def _literal_phase(g, c, l, w1, b1, w2, b2, scale):
    nblk = NUM_LITS // BL
    half = nblk // 2
    return pl.pallas_call(
        _literal_body,
        grid=(nblk,),
        in_specs=[
            pl.BlockSpec((NUM_CLAUSES, BL), lambda j: (0, j)),
            pl.BlockSpec((NUM_CLAUSES, D), lambda j: (0, 0)),
            pl.BlockSpec((BL, D), lambda j: (j, 0)),
            pl.BlockSpec((BL, D), lambda j: ((j + half) % nblk, 0)),
            pl.BlockSpec((3 * D, D), lambda j: (0, 0)),
            pl.BlockSpec((1, D), lambda j: (0, 0)),
            pl.BlockSpec((D, D), lambda j: (0, 0)),
            pl.BlockSpec((1, D), lambda j: (0, 0)),
            pl.BlockSpec(memory_space=pltpu.SMEM),
        ],
        out_specs=pl.BlockSpec((BL, D), lambda j: (j, 0)),
        out_shape=jax.ShapeDtypeStruct((NUM_LITS, D), jnp.float32),
    )(g, c, l, l, w1, b1, w2, b2, scale)


def _vote_phase(l, v_params):
    (w1, b1), (w2, b2), (w3, b3), (w4, b4) = v_params
    args = (l, w1, b1.reshape(1, D), w2, b2.reshape(1, D), w3,
            b3.reshape(1, D), w4, b4.reshape(1, 1))
    return pl.pallas_call(
        _vote_body,
        in_specs=[
            pl.BlockSpec(a.shape, functools.partial(lambda n: (0,) * n, a.ndim))
            for a in args
        ],
        out_specs=pl.BlockSpec((NUM_VARS, 1), lambda: (0, 0)),
        out_shape=jax.ShapeDtypeStruct((NUM_VARS, 1), jnp.float32),
    )(*args)


def kernel(G, c_params, l_params, v_params, c_init_scale, l_init_scale,
           cl_scale, lc_scale):
    L = jnp.full((NUM_LITS, D), 1.0, jnp.float32) * l_init_scale
    C = jnp.full((NUM_CLAUSES, D), 1.0, jnp.float32) * c_init_scale
    lc = lc_scale.reshape(1)
    cl = cl_scale.reshape(1)

    for r in range(NUM_ROUNDS):
        (w1c, b1c), (w2c, b2c) = c_params[r]
        C = _clause_phase(G, L, C, w1c, b1c.reshape(1, D),
                          w2c, b2c.reshape(1, D), lc)
        (w1l, b1l), (w2l, b2l) = l_params[r]
        L = _literal_phase(G, C, L, w1l, b1l.reshape(1, D),
                           w2l, b2l.reshape(1, D), cl)

    return _vote_phase(L, v_params).reshape(NUM_VARS)
